# Initial kernel scaffold; baseline (speedup 1.0000x reference)
import functools

import jax
import jax.numpy as jnp
from jax import lax
from jax.experimental import pallas as pl
from jax.experimental.pallas import tpu as pltpu
from jax.experimental.pallas import tpu_sc as plsc

N_NODES = 100000
N_EDGES = 6400000
C = 1.0
R_C = 1.0
GAMMA = 0.1
MIN_R = 0.1
EPS = 1e-12

NC = 2
NS = 16
NW = NC * NS
L = 16

G = 128
K = 16
CHUNK_E = K * G
NCHUNK = 98
GROUPS_PER_TILE = NCHUNK * K
NG = NW * GROUPS_PER_TILE
E_PAD = NG * G

ROWS_PER_TILE = 6272
N_PAD = NS * ROWS_PER_TILE


def _sc_body(x_hbm, srcg_hbm, dstg_hbm, zeros_hbm, out_hbm,
             acc_ref, src_v, dst_v, xs_v, xd_v, msg_v, sem_g, sem_s):
    c = lax.axis_index("c")
    s = lax.axis_index("s")
    w = c * NS + s

    rbase = s * ROWS_PER_TILE
    pltpu.sync_copy(zeros_hbm.at[pl.ds(rbase, ROWS_PER_TILE)],
                    acc_ref.at[pl.ds(rbase, ROWS_PER_TILE)])
    plsc.subcore_barrier()

    def chunk(cb, carry):
        gbase = w * GROUPS_PER_TILE + cb * K
        pltpu.sync_copy(srcg_hbm.at[pl.ds(gbase, K)], src_v)
        pltpu.sync_copy(dstg_hbm.at[pl.ds(gbase, K)], dst_v)

        descs = []
        for j in range(K):
            d = pltpu.make_async_copy(
                x_hbm.at[src_v.at[j]], xs_v.at[pl.ds(j * G, G)], sem_g)
            d.start()
            descs.append(d)
            d = pltpu.make_async_copy(
                x_hbm.at[dst_v.at[j]], xd_v.at[pl.ds(j * G, G)], sem_g)
            d.start()
            descs.append(d)
        for d in descs:
            d.wait()

        def compute(i, carry2):
            ridx = lax.iota(jnp.int32, L) + i * L
            col0 = jnp.zeros((L,), jnp.int32)
            col1 = jnp.ones((L,), jnp.int32)
            xs0 = plsc.load_gather(xs_v, [ridx, col0])
            xs1 = plsc.load_gather(xs_v, [ridx, col1])
            xd0 = plsc.load_gather(xd_v, [ridx, col0])
            xd1 = plsc.load_gather(xd_v, [ridx, col1])
            dx = xd0 - xs0
            dy = xd1 - xs1
            r2 = dx * dx + dy * dy
            yi = jnp.int32(0x5F3759DF) - lax.shift_right_logical(
                lax.bitcast_convert_type(r2, jnp.int32), 1)
            y = lax.bitcast_convert_type(yi, jnp.float32)
            h = 0.5 * r2
            y = y * (1.5 - h * y * y)
            y = y * (1.5 - h * y * y)
            y = y * (1.5 - h * y * y)
            r = r2 * y
            u = jnp.where(r > MIN_R, y, 1.0 / MIN_R)
            u2 = u * u
            u6 = u2 * u2 * u2
            f = (48.0 * C * u6 - 24.0 * C) * u6 * u
            inv = 1.0 / jnp.maximum(r, EPS)
            coef = f * inv
            plsc.store_scatter(msg_v, [ridx, col0], coef * dx)
            plsc.store_scatter(msg_v, [ridx, col1], coef * dy)
            return carry2

        lax.fori_loop(0, CHUNK_E // L, compute, 0)

        descs = []
        for j in range(K):
            d = pltpu.make_async_copy(
                msg_v.at[pl.ds(j * G, G)], acc_ref.at[dst_v.at[j]], sem_s)
            d.start(add=True)
            descs.append(d)
        for d in descs:
            d.wait()
        return carry

    lax.fori_loop(0, NCHUNK, chunk, 0)

    plsc.subcore_barrier()
    pltpu.sync_copy(acc_ref.at[pl.ds(rbase, ROWS_PER_TILE)],
                    out_hbm.at[c, pl.ds(rbase, ROWS_PER_TILE)])


_sc_kernel = functools.partial(
    pl.kernel,
    mesh=plsc.VectorSubcoreMesh(
        core_axis_name="c", subcore_axis_name="s",
        num_cores=NC, num_subcores=NS),
    out_type=jax.ShapeDtypeStruct((NC, N_PAD, 2), jnp.float32),
    scratch_types=[
        pltpu.VMEM_SHARED((N_PAD, 2), jnp.float32),
        pltpu.VMEM((K, G), jnp.int32),
        pltpu.VMEM((K, G), jnp.int32),
        pltpu.VMEM((CHUNK_E, 2), jnp.float32),
        pltpu.VMEM((CHUNK_E, 2), jnp.float32),
        pltpu.VMEM((CHUNK_E, 2), jnp.float32),
        pltpu.SemaphoreType.DMA,
        pltpu.SemaphoreType.DMA,
    ],
)(_sc_body)


def _combine_body(p_ref, v_ref, o_ref):
    o_ref[...] = p_ref[0] + p_ref[1] - GAMMA * v_ref[...]


def kernel(x, v, edge_index):
    src = edge_index[0]
    dst = edge_index[1]
    pad = E_PAD - N_EDGES
    srcg = jnp.concatenate([src, jnp.zeros((pad,), jnp.int32)]).reshape(NG, G)
    dstg = jnp.concatenate([dst, jnp.zeros((pad,), jnp.int32)]).reshape(NG, G)
    zeros = jnp.zeros((N_PAD, 2), jnp.float32)

    partial = _sc_kernel(x, srcg, dstg, zeros)

    p3 = partial.reshape(2, N_PAD * 2 // 128, 128)
    v3 = jnp.pad(v, ((0, N_PAD - N_NODES), (0, 0))).reshape(N_PAD * 2 // 128, 128)
    out = pl.pallas_call(
        _combine_body,
        out_shape=jax.ShapeDtypeStruct((N_PAD * 2 // 128, 128), jnp.float32),
    )(p3, v3)
    return out.reshape(N_PAD, 2)[:N_NODES]


# speedup vs baseline: 44.8688x; 44.8688x over previous
import functools

import jax
import jax.numpy as jnp
from jax import lax
from jax.experimental import pallas as pl
from jax.experimental.pallas import tpu as pltpu
from jax.experimental.pallas import tpu_sc as plsc

N_NODES = 100000
N_EDGES = 6400000
C = 1.0
R_C = 1.0
GAMMA = 0.1
MIN_R = 0.1
EPS = 1e-12

NC = 2
NS = 16
NW = NC * NS
L = 16

G = 128
K = 16
CHUNK_E = K * G
NCHUNK = 98
GROUPS_PER_TILE = NCHUNK * K
NG = NW * GROUPS_PER_TILE
E_PAD = NG * G

ROWS_PER_TILE = 6272
N_PAD = NS * ROWS_PER_TILE
NROW = N_PAD // 128


def _sc_body(x0_hbm, x1_hbm, srcg_hbm, dstg_hbm, zeros_hbm, out_hbm,
             acc0, acc1, src_v, dst_v, xs0_v, xs1_v, xd0_v, xd1_v,
             m0_v, m1_v, sem_g, sem_s):
    c = lax.axis_index("c")
    s = lax.axis_index("s")
    w = c * NS + s

    rbase = s * ROWS_PER_TILE
    pltpu.sync_copy(zeros_hbm.at[pl.ds(rbase, ROWS_PER_TILE)],
                    acc0.at[pl.ds(rbase, ROWS_PER_TILE)])
    pltpu.sync_copy(zeros_hbm.at[pl.ds(rbase, ROWS_PER_TILE)],
                    acc1.at[pl.ds(rbase, ROWS_PER_TILE)])
    plsc.subcore_barrier()

    def chunk(cb, carry):
        gbase = w * GROUPS_PER_TILE + cb * K
        pltpu.sync_copy(srcg_hbm.at[pl.ds(gbase, K)], src_v)
        pltpu.sync_copy(dstg_hbm.at[pl.ds(gbase, K)], dst_v)

        descs = []
        for j in range(K):
            sl = pl.ds(j * G, G)
            for tab, idx, dest in ((x0_hbm, src_v, xs0_v),
                                   (x1_hbm, src_v, xs1_v),
                                   (x0_hbm, dst_v, xd0_v),
                                   (x1_hbm, dst_v, xd1_v)):
                d = pltpu.make_async_copy(tab.at[idx.at[j]], dest.at[sl], sem_g)
                d.start()
                descs.append(d)
        for d in descs:
            d.wait()

        def compute(i, carry2):
            sl = pl.ds(i * L, L)
            dx = xd0_v[sl] - xs0_v[sl]
            dy = xd1_v[sl] - xs1_v[sl]
            r2 = dx * dx + dy * dy
            yi = jnp.int32(0x5F3759DF) - lax.shift_right_logical(
                lax.bitcast_convert_type(r2, jnp.int32), 1)
            y = lax.bitcast_convert_type(yi, jnp.float32)
            h = 0.5 * r2
            y = y * (1.5 - h * y * y)
            y = y * (1.5 - h * y * y)
            y = y * (1.5 - h * y * y)
            r = r2 * y
            u = jnp.where(r > MIN_R, y, 1.0 / MIN_R)
            u2 = u * u
            u6 = u2 * u2 * u2
            f = (48.0 * C * u6 - 24.0 * C) * u6 * u
            inv = 1.0 / jnp.maximum(r, EPS)
            coef = f * inv
            m0_v[sl] = coef * dx
            m1_v[sl] = coef * dy
            return carry2

        lax.fori_loop(0, CHUNK_E // L, compute, 0)

        descs = []
        for j in range(K):
            sl = pl.ds(j * G, G)
            d = pltpu.make_async_copy(m0_v.at[sl], acc0.at[dst_v.at[j]], sem_s)
            d.start(add=True)
            descs.append(d)
            d = pltpu.make_async_copy(m1_v.at[sl], acc1.at[dst_v.at[j]], sem_s)
            d.start(add=True)
            descs.append(d)
        for d in descs:
            d.wait()
        return carry

    lax.fori_loop(0, NCHUNK, chunk, 0)

    plsc.subcore_barrier()
    pltpu.sync_copy(acc0.at[pl.ds(rbase, ROWS_PER_TILE)],
                    out_hbm.at[c, 0, pl.ds(rbase, ROWS_PER_TILE)])
    pltpu.sync_copy(acc1.at[pl.ds(rbase, ROWS_PER_TILE)],
                    out_hbm.at[c, 1, pl.ds(rbase, ROWS_PER_TILE)])


_sc_kernel = functools.partial(
    pl.kernel,
    mesh=plsc.VectorSubcoreMesh(
        core_axis_name="c", subcore_axis_name="s",
        num_cores=NC, num_subcores=NS),
    out_type=jax.ShapeDtypeStruct((NC, 2, N_PAD), jnp.float32),
    scratch_types=[
        pltpu.VMEM_SHARED((N_PAD,), jnp.float32),
        pltpu.VMEM_SHARED((N_PAD,), jnp.float32),
        pltpu.VMEM((K, G), jnp.int32),
        pltpu.VMEM((K, G), jnp.int32),
        pltpu.VMEM((CHUNK_E,), jnp.float32),
        pltpu.VMEM((CHUNK_E,), jnp.float32),
        pltpu.VMEM((CHUNK_E,), jnp.float32),
        pltpu.VMEM((CHUNK_E,), jnp.float32),
        pltpu.VMEM((CHUNK_E,), jnp.float32),
        pltpu.VMEM((CHUNK_E,), jnp.float32),
        pltpu.SemaphoreType.DMA,
        pltpu.SemaphoreType.DMA,
    ],
)(_sc_body)


def _combine_body(p_ref, v_ref, o_ref):
    o_ref[0] = p_ref[0, 0] + p_ref[1, 0] - GAMMA * v_ref[0]
    o_ref[1] = p_ref[0, 1] + p_ref[1, 1] - GAMMA * v_ref[1]


def kernel(x, v, edge_index):
    x0 = x[:, 0]
    x1 = x[:, 1]
    src = edge_index[0]
    dst = edge_index[1]
    pad = E_PAD - N_EDGES
    srcg = jnp.concatenate([src, jnp.zeros((pad,), jnp.int32)]).reshape(NG, G)
    dstg = jnp.concatenate([dst, jnp.zeros((pad,), jnp.int32)]).reshape(NG, G)
    zeros = jnp.zeros((N_PAD,), jnp.float32)

    partial = _sc_kernel(x0, x1, srcg, dstg, zeros)

    p4 = partial.reshape(NC, 2, NROW, 128)
    v3 = jnp.pad(v.T, ((0, 0), (0, N_PAD - N_NODES))).reshape(2, NROW, 128)
    out = pl.pallas_call(
        _combine_body,
        out_shape=jax.ShapeDtypeStruct((2, NROW, 128), jnp.float32),
    )(p4, v3)
    return out.reshape(2, N_PAD)[:, :N_NODES].T
